# baseline (speedup 1.0000x reference)
import jax
import jax.numpy as jnp
from jax import lax
from jax.experimental import pallas as pl
from jax.experimental.pallas import tpu as pltpu
from jax.experimental.pallas import tpu_sc as plsc

N = 10000
E = 320000
D = 128
CK1 = 40
NCH1 = E // CK1
CK3 = 80
NCH3 = E // CK3
NC, NS = 2, 16
NW = NC * NS
N_PAD = 10112
ROWS_PER_SUB = N_PAD // NS
E_PAD = 327680
BE = 8192
BN = 2000


def _zero_rows(buf, nrows):
    def body(r, _):
        for j in range(D // 16):
            buf[r, pl.ds(j * 16, 16)] = jnp.zeros((16,), jnp.float32)
        return 0
    lax.fori_loop(0, nrows, body, 0)


def _slice_chunks(stage_rows):
    full = ROWS_PER_SUB // stage_rows
    out = [(i * stage_rows, stage_rows) for i in range(full)]
    rem = ROWS_PER_SUB - full * stage_rows
    if rem:
        out.append((full * stage_rows, rem))
    return out


def _zero_acc_slice(buf, acc, s, stage_rows):
    _zero_rows(buf, stage_rows)
    for off, ln in _slice_chunks(stage_rows):
        pltpu.sync_copy(buf.at[pl.ds(0, ln)],
                        acc.at[pl.ds(s * ROWS_PER_SUB + off, ln)])


def _readout_acc_slice(acc, out_hbm, c, s, stage_rows):
    for off, ln in _slice_chunks(stage_rows):
        sl = pl.ds(s * ROWS_PER_SUB + off, ln)
        pltpu.sync_copy(acc.at[sl], out_hbm.at[c, sl])


def _k1_body(h_hbm, src_hbm, dst_hbm,
             prod_hbm, sd_part_hbm, norms_hbm,
             idx_s0, idx_s1, idx_s2, idx_d0, idx_d1, idx_d2,
             hs0, hs1, hs2, hd0, hd1, hd2, nrm_v, acc,
             gs0, gs1, gs2, gd0, gd1, gd2,
             sc0, sc1, sc2, wr0, wr1, wr2):
    c = lax.axis_index("c")
    s = lax.axis_index("s")
    wid = s * NC + c
    idx_s = (idx_s0, idx_s1, idx_s2)
    idx_d = (idx_d0, idx_d1, idx_d2)
    hs = (hs0, hs1, hs2)
    hd = (hd0, hd1, hd2)
    sem_gs = (gs0, gs1, gs2)
    sem_gd = (gd0, gd1, gd2)
    sem_sc = (sc0, sc1, sc2)
    sem_wr = (wr0, wr1, wr2)

    _zero_acc_slice(hs0, acc, s, CK1)
    nrm_v[0, :] = jnp.zeros((16,), jnp.float32)
    nrm_v[1, :] = jnp.zeros((16,), jnp.float32)
    plsc.subcore_barrier()

    nw = NCH1 // NW

    def start_gather(t, b):
        base = (wid + t * NW) * CK1
        pltpu.sync_copy(src_hbm.at[pl.ds(base, CK1)], idx_s[b])
        pltpu.sync_copy(dst_hbm.at[pl.ds(base, CK1)], idx_d[b])
        pltpu.async_copy(h_hbm.at[idx_s[b]], hs[b], sem_gs[b])
        pltpu.async_copy(h_hbm.at[idx_d[b]], hd[b], sem_gd[b])

    def wait_writes(b):
        pltpu.make_async_copy(hs[b], acc.at[idx_d[b]], sem_sc[b]).wait()
        pltpu.make_async_copy(hd[b], prod_hbm.at[pl.ds(0, CK1)],
                              sem_wr[b]).wait()

    def process(t, b, wait_prev, start_next):
        bn = (b + 1) % 3
        if wait_prev:
            wait_writes(bn)
        if start_next:
            start_gather(t + 1, bn)
        base = (wid + t * NW) * CK1
        pltpu.make_async_copy(h_hbm.at[idx_s[b]], hs[b], sem_gs[b]).wait()
        pltpu.make_async_copy(h_hbm.at[idx_d[b]], hd[b], sem_gd[b]).wait()

        def row_body(r, rc):
            rns, rnd = rc
            for j in range(D // 16):
                ds = pl.ds(j * 16, 16)
                a = hs[b][r, ds]
                bb = hd[b][r, ds]
                hd[b][r, ds] = a * bb
                hs[b][r, ds] = a - bb
                rns = rns + a * a
                rnd = rnd + bb * bb
            return rns, rnd

        z16 = jnp.zeros((16,), jnp.float32)
        rns, rnd = lax.fori_loop(0, CK1, row_body, (z16, z16))
        nrm_v[0, :] = nrm_v[0, :] + rns
        nrm_v[1, :] = nrm_v[1, :] + rnd
        pltpu.async_copy(hs[b], acc.at[idx_d[b]], sem_sc[b], add=True)
        pltpu.async_copy(hd[b], prod_hbm.at[pl.ds(base, CK1)], sem_wr[b])

    start_gather(0, 0)
    process(0, 0, False, True)
    process(1, 1, False, True)

    def tri_body(i3, _):
        for k in range(3):
            t = 2 + i3 * 3 + k
            process(t, (2 + k) % 3, True, True)
        return 0

    lax.fori_loop(0, (nw - 4) // 3, tri_body, 0)
    process(nw - 2, (nw - 2) % 3, True, True)
    process(nw - 1, (nw - 1) % 3, True, False)
    wait_writes((nw - 2) % 3)
    wait_writes((nw - 1) % 3)

    pltpu.sync_copy(nrm_v, norms_hbm.at[:, wid])
    plsc.subcore_barrier()
    _readout_acc_slice(acc, sd_part_hbm, c, s, CK1)


def _k3_body(sd_hbm, src_hbm, dst_hbm, score_hbm,
             hdp_hbm,
             idx_s0, idx_s1, idx_s2, idx_d0, idx_d1, idx_d2,
             sv0, sv1, sv2, buf0, buf1, buf2, acc,
             g0, g1, g2, sc0, sc1, sc2):
    c = lax.axis_index("c")
    s = lax.axis_index("s")
    wid = s * NC + c
    idx_s = (idx_s0, idx_s1, idx_s2)
    idx_d = (idx_d0, idx_d1, idx_d2)
    sv = (sv0, sv1, sv2)
    buf = (buf0, buf1, buf2)
    sem_g = (g0, g1, g2)
    sem_sc = (sc0, sc1, sc2)

    _zero_acc_slice(buf0, acc, s, CK3)
    plsc.subcore_barrier()

    nw = NCH3 // NW

    def start_gather(t, b):
        base = (wid + t * NW) * CK3
        pltpu.sync_copy(src_hbm.at[pl.ds(base, CK3)], idx_s[b])
        pltpu.sync_copy(dst_hbm.at[pl.ds(base, CK3)], idx_d[b])
        pltpu.sync_copy(score_hbm.at[pl.ds(base, CK3)], sv[b])
        pltpu.async_copy(sd_hbm.at[idx_s[b]], buf[b], sem_g[b])

    def wait_scatter(b):
        pltpu.make_async_copy(buf[b], acc.at[idx_d[b]], sem_sc[b]).wait()

    def process(t, b, wait_prev, start_next):
        bn = (b + 1) % 3
        if wait_prev:
            wait_scatter(bn)
        if start_next:
            @pl.when(t + 1 < nw)
            def _():
                start_gather(t + 1, bn)
        pltpu.make_async_copy(sd_hbm.at[idx_s[b]], buf[b], sem_g[b]).wait()

        def row_body(r, _rc):
            grp = sv[b][pl.ds((r // 16) * 16, 16)]
            lane = jnp.full((16,), r % 16, jnp.int32)
            sval = lax.gather(
                grp, lane[:, None],
                lax.GatherDimensionNumbers(offset_dims=(),
                                           collapsed_slice_dims=(0,),
                                           start_index_map=(0,)),
                (1,), mode=lax.GatherScatterMode.PROMISE_IN_BOUNDS)
            for j in range(D // 16):
                ds = pl.ds(j * 16, 16)
                buf[b][r, ds] = buf[b][r, ds] * sval
            return 0

        lax.fori_loop(0, CK3, row_body, 0)
        pltpu.async_copy(buf[b], acc.at[idx_d[b]], sem_sc[b], add=True)

    start_gather(0, 0)
    process(0, 0, False, True)
    process(1, 1, False, True)

    def tri_body(i3, _):
        for k in range(3):
            t = 2 + i3 * 3 + k
            process(t, (2 + k) % 3, True, True)
        return 0

    lax.fori_loop(0, (nw - 2) // 3, tri_body, 0)
    wait_scatter((nw - 2) % 3)
    wait_scatter((nw - 1) % 3)

    plsc.subcore_barrier()
    _readout_acc_slice(acc, hdp_hbm, c, s, CK3)


def _score_body(norms_ref, prod_ref, p_ref, out_ref):
    nsq = jnp.sum(norms_ref[...], axis=1)
    scale = jnp.sqrt(nsq[0]) * jnp.sqrt(nsq[1]) + 1e-06
    x = prod_ref[...]
    xh = x.astype(jnp.bfloat16)
    xl = (x - xh.astype(jnp.float32)).astype(jnp.bfloat16)
    p = p_ref[...]
    ph = p.astype(jnp.bfloat16)
    pl_ = (p - ph.astype(jnp.float32)).astype(jnp.bfloat16)
    dn = (((0,), (1,)), ((), ()))
    t = (lax.dot_general(ph, xh, dn, preferred_element_type=jnp.float32)
         + lax.dot_general(pl_, xh, dn, preferred_element_type=jnp.float32)
         + lax.dot_general(ph, xl, dn, preferred_element_type=jnp.float32))
    t = jax.nn.relu(t)
    out_ref[...] = jnp.exp(jnp.clip(jnp.sum(t, axis=0) / scale, -5.0, 5.0))


def _combine_body(a_ref, out_ref):
    out_ref[...] = a_ref[0] + a_ref[1]


def _ffn_body(hp_ref, w_ref, b_ref, out_ref):
    x = hp_ref[0] + hp_ref[1]
    y = lax.dot_general(x, w_ref[...], (((1,), (1,)), ((), ())),
                        preferred_element_type=jnp.float32)
    out_ref[...] = jax.nn.relu(y + b_ref[...])


_sc_mesh = plsc.VectorSubcoreMesh(core_axis_name="c", subcore_axis_name="s")

_k1 = pl.kernel(
    _k1_body,
    out_type=[
        jax.ShapeDtypeStruct((E_PAD, D), jnp.float32),
        jax.ShapeDtypeStruct((NC, N_PAD, D), jnp.float32),
        jax.ShapeDtypeStruct((2, NW, 16), jnp.float32),
    ],
    mesh=_sc_mesh,
    scratch_types=(
        [pltpu.VMEM((CK1,), jnp.int32) for _ in range(6)]
        + [pltpu.VMEM((CK1, D), jnp.float32) for _ in range(6)]
        + [pltpu.VMEM((2, 16), jnp.float32),
           pltpu.VMEM_SHARED((N_PAD, D), jnp.float32)]
        + [pltpu.SemaphoreType.DMA for _ in range(12)]
    ),
    name="k1_gather_prod_segsum",
)

_k3 = pl.kernel(
    _k3_body,
    out_type=jax.ShapeDtypeStruct((NC, N_PAD, D), jnp.float32),
    mesh=_sc_mesh,
    scratch_types=(
        [pltpu.VMEM((CK3,), jnp.int32) for _ in range(6)]
        + [pltpu.VMEM((CK3,), jnp.float32) for _ in range(3)]
        + [pltpu.VMEM((CK3, D), jnp.float32) for _ in range(3)]
        + [pltpu.VMEM_SHARED((N_PAD, D), jnp.float32)]
        + [pltpu.SemaphoreType.DMA for _ in range(6)]
    ),
    name="k3_weighted_segsum",
)


def kernel(h, edge_index, proj_cosim, ffn_w, ffn_b):
    src = edge_index[0]
    dst = edge_index[1]

    prod, sd_part, norms = _k1(h, src, dst)

    score = pl.pallas_call(
        _score_body,
        grid=(E_PAD // BE,),
        in_specs=[
            pl.BlockSpec((2, NW * 16), lambda i: (0, 0)),
            pl.BlockSpec((BE, D), lambda i: (i, 0)),
            pl.BlockSpec((D, D), lambda i: (0, 0)),
        ],
        out_specs=pl.BlockSpec((BE,), lambda i: (i,)),
        out_shape=jax.ShapeDtypeStruct((E_PAD,), jnp.float32),
    )(norms.reshape(2, NW * 16), prod, proj_cosim)

    sd = pl.pallas_call(
        _combine_body,
        grid=(4,),
        in_specs=[pl.BlockSpec((NC, 2528, D), lambda i: (0, i, 0))],
        out_specs=pl.BlockSpec((2528, D), lambda i: (i, 0)),
        out_shape=jax.ShapeDtypeStruct((N_PAD, D), jnp.float32),
    )(sd_part)

    hd_part = _k3(sd, src, dst, score)

    out = pl.pallas_call(
        _ffn_body,
        grid=(N // BN,),
        in_specs=[
            pl.BlockSpec((NC, BN, D), lambda i: (0, i, 0)),
            pl.BlockSpec((D, D), lambda i: (0, 0)),
            pl.BlockSpec((1, D), lambda i: (0, 0)),
        ],
        out_specs=pl.BlockSpec((BN, D), lambda i: (i, 0)),
        out_shape=jax.ShapeDtypeStruct((N, D), jnp.float32),
    )(hd_part, ffn_w, ffn_b.reshape(1, D))

    return out


# speedup vs baseline: 1.0014x; 1.0014x over previous
import jax
import jax.numpy as jnp
from jax import lax
from jax.experimental import pallas as pl
from jax.experimental.pallas import tpu as pltpu
from jax.experimental.pallas import tpu_sc as plsc

N = 10000
E = 320000
D = 128
CK1 = 40
NCH1 = E // CK1
CK3 = 80
NCH3 = E // CK3
NC, NS = 2, 16
NW = NC * NS
N_PAD = 10112
ROWS_PER_SUB = N_PAD // NS
E_PAD = 327680
BE = 8192
BN = 2000


def _zero_rows(buf, nrows):
    def body(r, _):
        for j in range(D // 16):
            buf[r, pl.ds(j * 16, 16)] = jnp.zeros((16,), jnp.float32)
        return 0
    lax.fori_loop(0, nrows, body, 0)


def _slice_chunks(stage_rows):
    full = ROWS_PER_SUB // stage_rows
    out = [(i * stage_rows, stage_rows) for i in range(full)]
    rem = ROWS_PER_SUB - full * stage_rows
    if rem:
        out.append((full * stage_rows, rem))
    return out


def _zero_acc_slice(buf, acc, s, stage_rows):
    _zero_rows(buf, stage_rows)
    for off, ln in _slice_chunks(stage_rows):
        pltpu.sync_copy(buf.at[pl.ds(0, ln)],
                        acc.at[pl.ds(s * ROWS_PER_SUB + off, ln)])


def _readout_acc_slice(acc, out_hbm, c, s, stage_rows):
    for off, ln in _slice_chunks(stage_rows):
        sl = pl.ds(s * ROWS_PER_SUB + off, ln)
        pltpu.sync_copy(acc.at[sl], out_hbm.at[c, sl])


def _k1_body(h_hbm, src_hbm, dst_hbm,
             prod_hbm, sd_part_hbm, norms_hbm,
             idx_s0, idx_s1, idx_s2, idx_s3, idx_d0, idx_d1, idx_d2, idx_d3,
             hs0, hs1, hs2, hs3, hd0, hd1, hd2, hd3, nrm_v, acc,
             gs0, gs1, gs2, gs3, gd0, gd1, gd2, gd3,
             sc0, sc1, sc2, sc3, wr0, wr1, wr2, wr3):
    c = lax.axis_index("c")
    s = lax.axis_index("s")
    wid = s * NC + c
    idx_s = (idx_s0, idx_s1, idx_s2, idx_s3)
    idx_d = (idx_d0, idx_d1, idx_d2, idx_d3)
    hs = (hs0, hs1, hs2, hs3)
    hd = (hd0, hd1, hd2, hd3)
    sem_gs = (gs0, gs1, gs2, gs3)
    sem_gd = (gd0, gd1, gd2, gd3)
    sem_sc = (sc0, sc1, sc2, sc3)
    sem_wr = (wr0, wr1, wr2, wr3)

    _zero_acc_slice(hs0, acc, s, CK1)
    nrm_v[0, :] = jnp.zeros((16,), jnp.float32)
    nrm_v[1, :] = jnp.zeros((16,), jnp.float32)
    plsc.subcore_barrier()

    nw = NCH1 // NW

    def start_gather(t, b):
        base = (wid + t * NW) * CK1
        pltpu.sync_copy(src_hbm.at[pl.ds(base, CK1)], idx_s[b])
        pltpu.sync_copy(dst_hbm.at[pl.ds(base, CK1)], idx_d[b])
        pltpu.async_copy(h_hbm.at[idx_s[b]], hs[b], sem_gs[b])
        pltpu.async_copy(h_hbm.at[idx_d[b]], hd[b], sem_gd[b])

    def wait_writes(b):
        pltpu.make_async_copy(hs[b], acc.at[idx_d[b]], sem_sc[b]).wait()
        pltpu.make_async_copy(hd[b], prod_hbm.at[pl.ds(0, CK1)],
                              sem_wr[b]).wait()

    def process(t, b):
        bn = (b + 2) % 4

        @pl.when(t >= 2)
        def _():
            wait_writes(bn)

        @pl.when(t + 2 < nw)
        def _():
            start_gather(t + 2, bn)
        base = (wid + t * NW) * CK1
        pltpu.make_async_copy(h_hbm.at[idx_s[b]], hs[b], sem_gs[b]).wait()
        pltpu.make_async_copy(h_hbm.at[idx_d[b]], hd[b], sem_gd[b]).wait()

        def row_body(r, rc):
            rns, rnd = rc
            for j in range(D // 16):
                ds = pl.ds(j * 16, 16)
                a = hs[b][r, ds]
                bb = hd[b][r, ds]
                hd[b][r, ds] = a * bb
                hs[b][r, ds] = a - bb
                rns = rns + a * a
                rnd = rnd + bb * bb
            return rns, rnd

        z16 = jnp.zeros((16,), jnp.float32)
        rns, rnd = lax.fori_loop(0, CK1, row_body, (z16, z16))
        nrm_v[0, :] = nrm_v[0, :] + rns
        nrm_v[1, :] = nrm_v[1, :] + rnd
        pltpu.async_copy(hs[b], acc.at[idx_d[b]], sem_sc[b], add=True)
        pltpu.async_copy(hd[b], prod_hbm.at[pl.ds(base, CK1)], sem_wr[b])

    start_gather(0, 0)
    start_gather(1, 1)

    def quad_body(i4, _):
        for k in range(4):
            process(i4 * 4 + k, k)
        return 0

    lax.fori_loop(0, nw // 4, quad_body, 0)
    process(nw - 2, (nw - 2) % 4)
    process(nw - 1, (nw - 1) % 4)
    wait_writes((nw - 2) % 4)
    wait_writes((nw - 1) % 4)

    pltpu.sync_copy(nrm_v, norms_hbm.at[:, wid])
    plsc.subcore_barrier()
    _readout_acc_slice(acc, sd_part_hbm, c, s, CK1)


def _k3_body(sd_hbm, src_hbm, dst_hbm, score_hbm,
             hdp_hbm,
             idx_s0, idx_s1, idx_s2, idx_s3, idx_d0, idx_d1, idx_d2, idx_d3,
             sv0, sv1, sv2, sv3, buf0, buf1, buf2, buf3, acc,
             g0, g1, g2, g3, sc0, sc1, sc2, sc3):
    c = lax.axis_index("c")
    s = lax.axis_index("s")
    wid = s * NC + c
    idx_s = (idx_s0, idx_s1, idx_s2, idx_s3)
    idx_d = (idx_d0, idx_d1, idx_d2, idx_d3)
    sv = (sv0, sv1, sv2, sv3)
    buf = (buf0, buf1, buf2, buf3)
    sem_g = (g0, g1, g2, g3)
    sem_sc = (sc0, sc1, sc2, sc3)

    _zero_acc_slice(buf0, acc, s, CK3)
    plsc.subcore_barrier()

    nw = NCH3 // NW

    def start_gather(t, b):
        base = (wid + t * NW) * CK3
        pltpu.sync_copy(src_hbm.at[pl.ds(base, CK3)], idx_s[b])
        pltpu.sync_copy(dst_hbm.at[pl.ds(base, CK3)], idx_d[b])
        pltpu.sync_copy(score_hbm.at[pl.ds(base, CK3)], sv[b])
        pltpu.async_copy(sd_hbm.at[idx_s[b]], buf[b], sem_g[b])

    def wait_scatter(b):
        pltpu.make_async_copy(buf[b], acc.at[idx_d[b]], sem_sc[b]).wait()

    def process(t, b):
        bn = (b + 2) % 4

        @pl.when(t >= 2)
        def _():
            wait_scatter(bn)

        @pl.when(t + 2 < nw)
        def _():
            start_gather(t + 2, bn)
        pltpu.make_async_copy(sd_hbm.at[idx_s[b]], buf[b], sem_g[b]).wait()

        def row_body(r, _rc):
            grp = sv[b][pl.ds((r // 16) * 16, 16)]
            lane = jnp.full((16,), r % 16, jnp.int32)
            sval = lax.gather(
                grp, lane[:, None],
                lax.GatherDimensionNumbers(offset_dims=(),
                                           collapsed_slice_dims=(0,),
                                           start_index_map=(0,)),
                (1,), mode=lax.GatherScatterMode.PROMISE_IN_BOUNDS)
            for j in range(D // 16):
                ds = pl.ds(j * 16, 16)
                buf[b][r, ds] = buf[b][r, ds] * sval
            return 0

        lax.fori_loop(0, CK3, row_body, 0)
        pltpu.async_copy(buf[b], acc.at[idx_d[b]], sem_sc[b], add=True)

    start_gather(0, 0)
    start_gather(1, 1)

    def quad_body(i4, _):
        for k in range(4):
            process(i4 * 4 + k, k)
        return 0

    lax.fori_loop(0, nw // 4, quad_body, 0)
    process(nw - 1, (nw - 1) % 4)
    wait_scatter((nw - 2) % 4)
    wait_scatter((nw - 1) % 4)

    plsc.subcore_barrier()
    _readout_acc_slice(acc, hdp_hbm, c, s, CK3)


def _score_body(norms_ref, prod_ref, p_ref, out_ref):
    nsq = jnp.sum(norms_ref[...], axis=1)
    scale = jnp.sqrt(nsq[0]) * jnp.sqrt(nsq[1]) + 1e-06
    x = prod_ref[...]
    xh = x.astype(jnp.bfloat16)
    xl = (x - xh.astype(jnp.float32)).astype(jnp.bfloat16)
    p = p_ref[...]
    ph = p.astype(jnp.bfloat16)
    pl_ = (p - ph.astype(jnp.float32)).astype(jnp.bfloat16)
    dn = (((0,), (1,)), ((), ()))
    t = (lax.dot_general(ph, xh, dn, preferred_element_type=jnp.float32)
         + lax.dot_general(pl_, xh, dn, preferred_element_type=jnp.float32)
         + lax.dot_general(ph, xl, dn, preferred_element_type=jnp.float32))
    t = jax.nn.relu(t)
    out_ref[...] = jnp.exp(jnp.clip(jnp.sum(t, axis=0) / scale, -5.0, 5.0))


def _combine_body(a_ref, out_ref):
    out_ref[...] = a_ref[0] + a_ref[1]


def _ffn_body(hp_ref, w_ref, b_ref, out_ref):
    x = hp_ref[0] + hp_ref[1]
    y = lax.dot_general(x, w_ref[...], (((1,), (1,)), ((), ())),
                        preferred_element_type=jnp.float32)
    out_ref[...] = jax.nn.relu(y + b_ref[...])


_sc_mesh = plsc.VectorSubcoreMesh(core_axis_name="c", subcore_axis_name="s")

_k1 = pl.kernel(
    _k1_body,
    out_type=[
        jax.ShapeDtypeStruct((E_PAD, D), jnp.float32),
        jax.ShapeDtypeStruct((NC, N_PAD, D), jnp.float32),
        jax.ShapeDtypeStruct((2, NW, 16), jnp.float32),
    ],
    mesh=_sc_mesh,
    scratch_types=(
        [pltpu.VMEM((CK1,), jnp.int32) for _ in range(8)]
        + [pltpu.VMEM((CK1, D), jnp.float32) for _ in range(8)]
        + [pltpu.VMEM((2, 16), jnp.float32),
           pltpu.VMEM_SHARED((N_PAD, D), jnp.float32)]
        + [pltpu.SemaphoreType.DMA for _ in range(16)]
    ),
    name="k1_gather_prod_segsum",
)

_k3 = pl.kernel(
    _k3_body,
    out_type=jax.ShapeDtypeStruct((NC, N_PAD, D), jnp.float32),
    mesh=_sc_mesh,
    scratch_types=(
        [pltpu.VMEM((CK3,), jnp.int32) for _ in range(8)]
        + [pltpu.VMEM((CK3,), jnp.float32) for _ in range(4)]
        + [pltpu.VMEM((CK3, D), jnp.float32) for _ in range(4)]
        + [pltpu.VMEM_SHARED((N_PAD, D), jnp.float32)]
        + [pltpu.SemaphoreType.DMA for _ in range(8)]
    ),
    name="k3_weighted_segsum",
)


def kernel(h, edge_index, proj_cosim, ffn_w, ffn_b):
    src = edge_index[0]
    dst = edge_index[1]

    prod, sd_part, norms = _k1(h, src, dst)

    score = pl.pallas_call(
        _score_body,
        grid=(E_PAD // BE,),
        in_specs=[
            pl.BlockSpec((2, NW * 16), lambda i: (0, 0)),
            pl.BlockSpec((BE, D), lambda i: (i, 0)),
            pl.BlockSpec((D, D), lambda i: (0, 0)),
        ],
        out_specs=pl.BlockSpec((BE,), lambda i: (i,)),
        out_shape=jax.ShapeDtypeStruct((E_PAD,), jnp.float32),
    )(norms.reshape(2, NW * 16), prod, proj_cosim)

    sd = pl.pallas_call(
        _combine_body,
        grid=(4,),
        in_specs=[pl.BlockSpec((NC, 2528, D), lambda i: (0, i, 0))],
        out_specs=pl.BlockSpec((2528, D), lambda i: (i, 0)),
        out_shape=jax.ShapeDtypeStruct((N_PAD, D), jnp.float32),
    )(sd_part)

    hd_part = _k3(sd, src, dst, score)

    out = pl.pallas_call(
        _ffn_body,
        grid=(N // BN,),
        in_specs=[
            pl.BlockSpec((NC, BN, D), lambda i: (0, i, 0)),
            pl.BlockSpec((D, D), lambda i: (0, 0)),
            pl.BlockSpec((1, D), lambda i: (0, 0)),
        ],
        out_specs=pl.BlockSpec((BN, D), lambda i: (i, 0)),
        out_shape=jax.ShapeDtypeStruct((N, D), jnp.float32),
    )(hd_part, ffn_w, ffn_b.reshape(1, D))

    return out
